# speedup vs baseline: 1.1136x; 1.1136x over previous
import functools

import jax
import jax.numpy as jnp
from jax import lax
from jax.experimental import pallas as pl
from jax.experimental.pallas import tpu as pltpu
from jax.experimental.pallas import tpu_sc as plsc

_BATCH = 16384

_info = plsc.get_sparse_core_info()
_NC = _info.num_cores
_NS = _info.num_subcores
_NW = _NC * _NS
_B_PER_W = _BATCH // _NW

_mesh = plsc.VectorSubcoreMesh(core_axis_name="c", subcore_axis_name="s")


@functools.partial(
    pl.kernel,
    mesh=_mesh,
    out_type=jax.ShapeDtypeStruct((_BATCH,), jnp.float32),
    scratch_types=[
        pltpu.VMEM((_B_PER_W,), jnp.int32),
        pltpu.VMEM((_B_PER_W,), jnp.float32),
        pltpu.SemaphoreType.DMA,
    ],
)
def _gather_sc(items_hbm, scores_hbm, out_hbm, idx_v, vals_v, sem):
    wid = lax.axis_index("s") * _NC + lax.axis_index("c")
    base = wid * _B_PER_W
    pltpu.sync_copy(items_hbm.at[pl.ds(base, _B_PER_W)], idx_v)
    pltpu.async_copy(scores_hbm.at[idx_v], vals_v, sem).wait()
    pltpu.sync_copy(vals_v, out_hbm.at[pl.ds(base, _B_PER_W)])


def kernel(users, items, scores):
    del users
    return _gather_sc(items, scores)


# speedup vs baseline: 1.1291x; 1.0139x over previous
import functools

import jax
import jax.numpy as jnp
from jax import lax
from jax.experimental import pallas as pl
from jax.experimental.pallas import tpu as pltpu
from jax.experimental.pallas import tpu_sc as plsc

_BATCH = 16384

_info = plsc.get_sparse_core_info()
_NC = _info.num_cores
_NS = _info.num_subcores
_NW = _NC * _NS
_B_PER_W = _BATCH // _NW

_mesh = plsc.VectorSubcoreMesh(core_axis_name="c", subcore_axis_name="s")


_HALF = _B_PER_W // 2


@functools.partial(
    pl.kernel,
    mesh=_mesh,
    out_type=jax.ShapeDtypeStruct((_BATCH,), jnp.float32),
    scratch_types=[
        pltpu.VMEM((_B_PER_W,), jnp.int32),
        pltpu.VMEM((_HALF,), jnp.float32),
        pltpu.VMEM((_HALF,), jnp.float32),
        pltpu.SemaphoreType.DMA,
        pltpu.SemaphoreType.DMA,
    ],
)
def _gather_sc(items_hbm, scores_hbm, out_hbm, idx_v, vals0, vals1, sem0, sem1):
    wid = lax.axis_index("s") * _NC + lax.axis_index("c")
    base = wid * _B_PER_W
    pltpu.sync_copy(items_hbm.at[pl.ds(base, _B_PER_W)], idx_v)
    g0 = pltpu.async_copy(scores_hbm.at[idx_v.at[pl.ds(0, _HALF)]], vals0, sem0)
    g1 = pltpu.async_copy(scores_hbm.at[idx_v.at[pl.ds(_HALF, _HALF)]], vals1, sem1)
    g0.wait()
    o0 = pltpu.async_copy(vals0, out_hbm.at[pl.ds(base, _HALF)], sem0)
    g1.wait()
    o1 = pltpu.async_copy(vals1, out_hbm.at[pl.ds(base + _HALF, _HALF)], sem1)
    o0.wait()
    o1.wait()


def kernel(users, items, scores):
    del users
    return _gather_sc(items, scores)
